# baseline (speedup 1.0000x reference)
import functools

import jax
import jax.numpy as jnp
from jax import lax
from jax.experimental import pallas as pl
from jax.experimental.pallas import tpu as pltpu
from jax.experimental.pallas import tpu_sc as plsc

N_TOT = 8388608
NC = 2
NS = 16
L = 16
NW = NC * NS
NE = N_TOT // NW
CH = 16384
NCH = NE // CH
NBUF = 3
U = 8
STEPS = CH // (U * L)

_mesh = plsc.VectorSubcoreMesh(core_axis_name="c", subcore_axis_name="s")


@functools.partial(
    pl.kernel,
    mesh=_mesh,
    out_type=jax.ShapeDtypeStruct((NW, 3, L), jnp.int32),
    scratch_types=(
        [pltpu.VMEM((CH,), jnp.int32) for _ in range(2 * NBUF)]
        + [pltpu.VMEM((3, L), jnp.int32)]
        + [pltpu.SemaphoreType.DMA for _ in range(NBUF)]
    ),
)
def _sc_partial_counts(p_hbm, g_hbm, out_hbm, *scr):
    pbufs = scr[0:NBUF]
    gbufs = scr[NBUF:2 * NBUF]
    accv = scr[2 * NBUF]
    sems = scr[2 * NBUF + 1:]

    wid = lax.axis_index("s") * NC + lax.axis_index("c")
    base = wid * NE

    def start(c):
        b = c % NBUF
        off = base + c * CH
        hp = pltpu.async_copy(p_hbm.at[pl.ds(off, CH)], pbufs[b], sems[b])
        hg = pltpu.async_copy(g_hbm.at[pl.ds(off, CH)], gbufs[b], sems[b])
        return hp, hg

    inflight = {c: start(c) for c in range(NBUF - 1)}
    zero = jnp.zeros((L,), jnp.int32)
    acc = (zero, zero, zero, zero, zero, zero)

    for c in range(NCH):
        if c + NBUF - 1 < NCH:
            inflight[c + NBUF - 1] = start(c + NBUF - 1)
        hp, hg = inflight.pop(c)
        hp.wait()
        hg.wait()
        b = c % NBUF
        pb = pbufs[b]
        gb = gbufs[b]

        @plsc.parallel_loop(0, STEPS, step=1, unroll=4, carry=acc)
        def acc(i, carry, pb=pb, gb=gb):
            a0, a1, a2, b0, b1, b2 = carry
            o0 = i * (U * L)
            for u in range(U):
                pv = pb[pl.ds(o0 + u * L, L)]
                gv = gb[pl.ds(o0 + u * L, L)]
                if u % 2 == 0:
                    a0 = a0 + pv
                    a1 = a1 + gv
                    a2 = a2 + (pv & gv)
                else:
                    b0 = b0 + pv
                    b1 = b1 + gv
                    b2 = b2 + (pv & gv)
            return a0, a1, a2, b0, b1, b2

    a0, a1, a2, b0, b1, b2 = acc
    accv[0] = a0 + b0
    accv[1] = a1 + b1
    accv[2] = a2 + b2
    pltpu.sync_copy(accv, out_hbm.at[wid])


def _combine_body(part_ref, conf_ref, out_ref):
    x = part_ref[...]
    s2 = jnp.sum(x, axis=0)
    s = jnp.sum(s2, axis=1)
    spf = s[0].astype(jnp.float32)
    sgf = s[1].astype(jnp.float32)
    spgf = s[2].astype(jnp.float32)
    c00 = jnp.float32(N_TOT) - spf - sgf + spgf
    c01 = sgf - spgf
    c10 = spf - spgf
    c11 = spgf
    ii = lax.broadcasted_iota(jnp.int32, (2, 2), 0)
    jj = lax.broadcasted_iota(jnp.int32, (2, 2), 1)
    cm = jnp.where(
        (ii == 0) & (jj == 0),
        c00,
        jnp.where((ii == 0) & (jj == 1), c01, jnp.where(jj == 0, c10, c11)),
    )
    out_ref[...] = conf_ref[...] + cm


_combine = pl.pallas_call(
    _combine_body,
    out_shape=jax.ShapeDtypeStruct((2, 2), jnp.float32),
)


def kernel(pred_labels, gt_labels, conf_matrix):
    partials = _sc_partial_counts(pred_labels, gt_labels)
    return _combine(partials, conf_matrix)


# speedup vs baseline: 1.0125x; 1.0125x over previous
import functools

import jax
import jax.numpy as jnp
from jax import lax
from jax.experimental import pallas as pl
from jax.experimental.pallas import tpu as pltpu
from jax.experimental.pallas import tpu_sc as plsc

N_TOT = 8388608
NC = 2
NS = 16
L = 16
NW = NC * NS
NE = N_TOT // NW
CH = 16384
NCH = NE // CH
NBUF = 3
U = 4
STEPS = CH // (U * L)

_mesh = plsc.VectorSubcoreMesh(core_axis_name="c", subcore_axis_name="s")


@functools.partial(
    pl.kernel,
    mesh=_mesh,
    out_type=jax.ShapeDtypeStruct((NW, 3, L), jnp.int32),
    scratch_types=(
        [pltpu.VMEM((CH,), jnp.int32) for _ in range(2 * NBUF)]
        + [pltpu.VMEM((3, L), jnp.int32)]
        + [pltpu.SemaphoreType.DMA for _ in range(NBUF)]
    ),
)
def _sc_partial_counts(p_hbm, g_hbm, out_hbm, *scr):
    pbufs = scr[0:NBUF]
    gbufs = scr[NBUF:2 * NBUF]
    accv = scr[2 * NBUF]
    sems = scr[2 * NBUF + 1:]

    wid = lax.axis_index("s") * NC + lax.axis_index("c")
    base = wid * NE

    def start(c):
        b = c % NBUF
        off = base + c * CH
        hp = pltpu.async_copy(p_hbm.at[pl.ds(off, CH)], pbufs[b], sems[b])
        hg = pltpu.async_copy(g_hbm.at[pl.ds(off, CH)], gbufs[b], sems[b])
        return hp, hg

    inflight = {c: start(c) for c in range(NBUF - 1)}
    zero = jnp.zeros((L,), jnp.int32)
    acc = (zero, zero, zero, zero, zero, zero)

    for c in range(NCH):
        if c + NBUF - 1 < NCH:
            inflight[c + NBUF - 1] = start(c + NBUF - 1)
        hp, hg = inflight.pop(c)
        hp.wait()
        hg.wait()
        b = c % NBUF
        pb = pbufs[b]
        gb = gbufs[b]

        @plsc.parallel_loop(0, STEPS, step=1, unroll=8, carry=acc)
        def acc(i, carry, pb=pb, gb=gb):
            a0, a1, a2, b0, b1, b2 = carry
            o0 = i * (U * L)
            for u in range(U):
                pv = pb[pl.ds(o0 + u * L, L)]
                gv = gb[pl.ds(o0 + u * L, L)]
                if u % 2 == 0:
                    a0 = a0 + pv
                    a1 = a1 + gv
                    a2 = a2 + (pv & gv)
                else:
                    b0 = b0 + pv
                    b1 = b1 + gv
                    b2 = b2 + (pv & gv)
            return a0, a1, a2, b0, b1, b2

    a0, a1, a2, b0, b1, b2 = acc
    accv[0] = a0 + b0
    accv[1] = a1 + b1
    accv[2] = a2 + b2
    pltpu.sync_copy(accv, out_hbm.at[wid])


def _combine_body(part_ref, conf_ref, out_ref):
    x = part_ref[...]
    s2 = jnp.sum(x, axis=0)
    s = jnp.sum(s2, axis=1)
    spf = s[0].astype(jnp.float32)
    sgf = s[1].astype(jnp.float32)
    spgf = s[2].astype(jnp.float32)
    c00 = jnp.float32(N_TOT) - spf - sgf + spgf
    c01 = sgf - spgf
    c10 = spf - spgf
    c11 = spgf
    ii = lax.broadcasted_iota(jnp.int32, (2, 2), 0)
    jj = lax.broadcasted_iota(jnp.int32, (2, 2), 1)
    cm = jnp.where(
        (ii == 0) & (jj == 0),
        c00,
        jnp.where((ii == 0) & (jj == 1), c01, jnp.where(jj == 0, c10, c11)),
    )
    out_ref[...] = conf_ref[...] + cm


_combine = pl.pallas_call(
    _combine_body,
    out_shape=jax.ShapeDtypeStruct((2, 2), jnp.float32),
)


def kernel(pred_labels, gt_labels, conf_matrix):
    partials = _sc_partial_counts(pred_labels, gt_labels)
    return _combine(partials, conf_matrix)
